# baseline (speedup 1.0000x reference)
import jax
import jax.numpy as jnp
from jax import lax
from jax.experimental import pallas as pl
from jax.experimental.pallas import tpu as pltpu
from jax.experimental.pallas import tpu_sc as plsc

_NUM_GRAPHS = 16
_TOTAL_NODES = 32768
_HIDDEN = 64
_LANES = 16
_NSUB = 16
_CHUNK = _TOTAL_NODES // _NSUB


def _body(nodes_hbm, att_hbm, batch_hbm, out_hbm,
          chunk_v, cnt_v, att_v, mat_v, row_v, shared_v, sem, att_sem):
    s = lax.axis_index("s")

    att_copy = pltpu.async_copy(att_hbm, att_v, att_sem)

    pltpu.sync_copy(batch_hbm.at[pl.ds(s * _CHUNK, _CHUNK)], chunk_v)

    g = lax.iota(jnp.int32, _LANES)
    lo = jnp.zeros((_LANES,), jnp.int32)
    step = _CHUNK // 2
    while step >= 1:
        t = lo + step
        val = plsc.load_gather(chunk_v, [t - 1])
        lo = jnp.where(val < g, t, lo)
        step //= 2
    val = plsc.load_gather(chunk_v, [lo])
    lo = lo + (val < g).astype(jnp.int32)
    cnt_v[...] = lo

    pltpu.sync_copy(cnt_v, shared_v.at[pl.ds(s * _LANES, _LANES)])
    plsc.subcore_barrier()

    pltpu.sync_copy(shared_v, mat_v)
    offsets = mat_v[pl.ds(0, _LANES)]
    for i in range(1, _NSUB):
        offsets = offsets + mat_v[pl.ds(i * _LANES, _LANES)]
    att_copy.wait()
    gidx = offsets + att_v[...]

    r = jnp.sum(jnp.where(g == s, gidx, 0))
    pltpu.async_copy(nodes_hbm.at[pl.ds(r, 1)], out_hbm.at[pl.ds(s, 1)],
                     sem).wait()


@jax.jit
def kernel(node_representations, attachment_indices, batch_indices):
    f = pl.kernel(
        _body,
        out_type=jax.ShapeDtypeStruct((_NUM_GRAPHS, _HIDDEN), jnp.float32),
        mesh=plsc.VectorSubcoreMesh(
            core_axis_name="c", subcore_axis_name="s", num_cores=1),
        compiler_params=pltpu.CompilerParams(
            needs_layout_passes=False,
            disable_bounds_checks=True,
            disable_semaphore_checks=True,
            skip_device_barrier=True,
        ),
        scratch_types=[
            pltpu.VMEM((_CHUNK,), jnp.int32),
            pltpu.VMEM((_LANES,), jnp.int32),
            pltpu.VMEM((_LANES,), jnp.int32),
            pltpu.VMEM((_NSUB * _LANES,), jnp.int32),
            pltpu.VMEM((1, _HIDDEN), jnp.float32),
            pltpu.VMEM_SHARED((_NSUB * _LANES,), jnp.int32),
            pltpu.SemaphoreType.DMA,
            pltpu.SemaphoreType.DMA,
        ],
    )
    return f(node_representations, attachment_indices, batch_indices)


# speedup vs baseline: 1.0171x; 1.0171x over previous
import jax
import jax.numpy as jnp
from jax import lax
from jax.experimental import pallas as pl
from jax.experimental.pallas import tpu as pltpu
from jax.experimental.pallas import tpu_sc as plsc

_NUM_GRAPHS = 16
_TOTAL_NODES = 32768
_HIDDEN = 64
_LANES = 16
_NSUB = 16
_CHUNK = _TOTAL_NODES // _NSUB


def _body(nodes_hbm, att_hbm, batch_hbm, out_hbm,
          chunk_v, cnt_v, att_v, mat_v, row_v, shared_v, sem, att_sem):
    s = lax.axis_index("s")

    att_copy = pltpu.async_copy(att_hbm, att_v, att_sem)

    pltpu.sync_copy(batch_hbm.at[pl.ds(s * _CHUNK, _CHUNK)], chunk_v)

    g = lax.iota(jnp.int32, _LANES)
    lo = jnp.zeros((_LANES,), jnp.int32)
    step = _CHUNK // 2
    while step >= 1:
        t = lo + step
        val = plsc.load_gather(chunk_v, [t - 1])
        lo = jnp.where(val < g, t, lo)
        step //= 2
    val = plsc.load_gather(chunk_v, [lo])
    lo = lo + (val < g).astype(jnp.int32)
    cnt_v[...] = lo

    pltpu.sync_copy(cnt_v, shared_v.at[pl.ds(s * _LANES, _LANES)])
    plsc.subcore_barrier()

    pltpu.sync_copy(shared_v, mat_v)
    offsets = mat_v[pl.ds(0, _LANES)]
    for i in range(1, _NSUB):
        offsets = offsets + mat_v[pl.ds(i * _LANES, _LANES)]
    att_copy.wait()
    gidx = jnp.minimum(offsets + att_v[...], _TOTAL_NODES - 1)

    r = jnp.sum(jnp.where(g == s, gidx, 0))
    pltpu.async_copy(nodes_hbm.at[pl.ds(r, 1)], row_v, sem).wait()
    pltpu.sync_copy(row_v, out_hbm.at[pl.ds(s, 1)])


@jax.jit
def kernel(node_representations, attachment_indices, batch_indices):
    f = pl.kernel(
        _body,
        out_type=jax.ShapeDtypeStruct((_NUM_GRAPHS, _HIDDEN), jnp.float32),
        mesh=plsc.VectorSubcoreMesh(
            core_axis_name="c", subcore_axis_name="s", num_cores=1),
        compiler_params=pltpu.CompilerParams(
            needs_layout_passes=False,
            disable_bounds_checks=True,
            disable_semaphore_checks=True,
            skip_device_barrier=True,
        ),
        scratch_types=[
            pltpu.VMEM((_CHUNK,), jnp.int32),
            pltpu.VMEM((_LANES,), jnp.int32),
            pltpu.VMEM((_LANES,), jnp.int32),
            pltpu.VMEM((_NSUB * _LANES,), jnp.int32),
            pltpu.VMEM((1, _HIDDEN), jnp.float32),
            pltpu.VMEM_SHARED((_NSUB * _LANES,), jnp.int32),
            pltpu.SemaphoreType.DMA,
            pltpu.SemaphoreType.DMA,
        ],
    )
    return f(node_representations, attachment_indices, batch_indices)
